# baseline (speedup 1.0000x reference)
import functools
import math

import numpy as np
import jax
import jax.numpy as jnp
from jax import lax
from jax.experimental import pallas as pl
from jax.experimental.pallas import tpu as pltpu
from jax.experimental.pallas import tpu_sc as plsc

_NUM_HEADS = 16
_REGION_NUM = 8
_ALPHA = 1.9
_BETA = 1.9 * 4
_GAMMA = 1.9 * 6

_NC = 2
_NS = 16
_NW = _NC * _NS


def _abs_piecewise_lut() -> np.ndarray:
    lut = np.zeros(256, np.int32)
    scale = (_BETA - 2 * _ALPHA) / math.log(_GAMMA / _ALPHA)
    for a in range(0, 128):
        if a == 0:
            v = 0
        elif a * 1.0 <= _ALPHA * 2:
            v = 1
        else:
            v = int(min(round(math.log(a / _ALPHA) * scale), 16.0))
        lut[127 + a] = v
        lut[127 - a] = v
    return lut


_ABS_LUT = _abs_piecewise_lut()
_TABLE_ROWS = 33


_NB = 8


def _make_sc_kernel(num_windows: int):
    win_per_tile = num_windows // _NW
    region = 16
    num_batches = win_per_tile // _NB

    mesh = plsc.VectorSubcoreMesh(core_axis_name="c", subcore_axis_name="s")

    @functools.partial(
        pl.kernel,
        out_type=jax.ShapeDtypeStruct(
            (num_windows, _NUM_HEADS, region, region), jnp.float32
        ),
        mesh=mesh,
        compiler_params=pltpu.CompilerParams(
            use_tc_tiling_on_sc=False, needs_layout_passes=False
        ),
        scratch_types=[
            pltpu.VMEM((win_per_tile * 2 * region,), jnp.float32),
            pltpu.VMEM((_NUM_HEADS * _TABLE_ROWS,), jnp.float32),
            pltpu.VMEM((256,), jnp.int32),
            pltpu.VMEM((2, _NB, _NUM_HEADS, region, region), jnp.float32),
            pltpu.SemaphoreType.DMA,
            pltpu.SemaphoreType.DMA,
        ],
    )
    def sc_kernel(
        cw_hbm, tf_hbm, lut_hbm, out_hbm, cw_v, tf_v, lut_v, out_v, sem0, sem1
    ):
        wid = lax.axis_index("s") * _NC + lax.axis_index("c")
        base = wid * win_per_tile
        pltpu.sync_copy(cw_hbm.at[pl.ds(base * 2 * region, win_per_tile * 2 * region)], cw_v)
        pltpu.sync_copy(tf_hbm, tf_v)
        pltpu.sync_copy(lut_hbm, lut_v)
        sems = (sem0, sem1)
        jiota = lax.iota(jnp.int32, region)

        def batch_pair(k, carry):
            for p in range(2):
                b = k * 2 + p

                @pl.when(k > 0)
                def _wait_prev():
                    pltpu.make_async_copy(
                        out_v.at[p], out_hbm.at[pl.ds(base, _NB)], sems[p]
                    ).wait()

                def _window(wloc, wcarry):
                    t = wloc - b * _NB
                    cbase = wloc * (2 * region) + 2 * jiota
                    c0 = plsc.load_gather(cw_v, [cbase])
                    c1 = plsc.load_gather(cw_v, [cbase + 1])
                    for i in range(region):
                        d0 = (c0[i] - c0).astype(jnp.int32) + 127
                        d1 = (c1[i] - c1).astype(jnp.int32) + 127
                        a0 = plsc.load_gather(lut_v, [d0])
                        a1 = plsc.load_gather(lut_v, [d1])
                        key = a0 + a1
                        for h in range(_NUM_HEADS):
                            row = plsc.load_gather(tf_v, [key + (h * _TABLE_ROWS)])
                            out_v[p, t, h, i, :] = row
                    return wcarry

                lax.fori_loop(b * _NB, (b + 1) * _NB, _window, 0)
                pltpu.async_copy(
                    out_v.at[p], out_hbm.at[pl.ds(base + b * _NB, _NB)], sems[p]
                )
            return carry

        lax.fori_loop(0, num_batches // 2, batch_pair, 0)
        for p in range(2):
            pltpu.make_async_copy(
                out_v.at[p], out_hbm.at[pl.ds(base, _NB)], sems[p]
            ).wait()

    return sc_kernel


def kernel(x, coords, relative_position_bias_table):
    B, Lc, _ = coords.shape
    H = int(np.ceil(np.sqrt(Lc)))
    H = H + ((-H) % _REGION_NUM)
    region = H // _REGION_NUM
    add_length = H * H - Lc
    if add_length > 0:
        coords = jnp.concatenate(
            [coords, jnp.zeros((B, add_length, 2), dtype=coords.dtype)], axis=1
        )
    num_windows = (B * H * H) // region

    cw = coords.reshape(-1)
    tf = relative_position_bias_table[:_TABLE_ROWS].T.reshape(-1)
    lut = jnp.asarray(_ABS_LUT)

    return _make_sc_kernel(num_windows)(cw, tf, lut)


# speedup vs baseline: 3.8785x; 3.8785x over previous
import functools
import math

import numpy as np
import jax
import jax.numpy as jnp
from jax import lax
from jax.experimental import pallas as pl
from jax.experimental.pallas import tpu as pltpu
from jax.experimental.pallas import tpu_sc as plsc

_NUM_HEADS = 16
_REGION_NUM = 8
_ALPHA = 1.9
_BETA = 1.9 * 4
_GAMMA = 1.9 * 6

_NC = 2
_NS = 16
_NW = _NC * _NS
_R = 16
_TABLE_ROWS = 33


def _abs_piecewise_lut() -> np.ndarray:
    lut = np.zeros(256, np.int32)
    scale = (_BETA - 2 * _ALPHA) / math.log(_GAMMA / _ALPHA)
    for a in range(0, 128):
        if a == 0:
            v = 0
        elif a * 1.0 <= _ALPHA * 2:
            v = 1
        else:
            v = int(min(round(math.log(a / _ALPHA) * scale), 16.0))
        lut[127 + a] = v
        lut[127 - a] = v
    return lut


_ABS_LUT = _abs_piecewise_lut()


def _make_sc_kernel(num_windows: int):
    win_per_tile = num_windows // _NW
    n_groups = win_per_tile // _R
    n_tw = num_windows // 128
    hh = _NUM_HEADS // 2

    mesh = plsc.VectorSubcoreMesh(core_axis_name="c", subcore_axis_name="s")

    @functools.partial(
        pl.kernel,
        out_type=jax.ShapeDtypeStruct(
            (_NUM_HEADS, _R, 2, n_tw, 8, 128), jnp.float32
        ),
        mesh=mesh,
        compiler_params=pltpu.CompilerParams(
            use_tc_tiling_on_sc=False, needs_layout_passes=False
        ),
        scratch_types=[
            pltpu.VMEM((win_per_tile * 2 * _R,), jnp.float32),
            pltpu.VMEM((_NUM_HEADS * _TABLE_ROWS,), jnp.float32),
            pltpu.VMEM((256,), jnp.int32),
            pltpu.VMEM((_R, _R, _R), jnp.int32),
            pltpu.VMEM((2, hh, _R, 2, 8, _R), jnp.float32),
            pltpu.SemaphoreType.DMA,
            pltpu.SemaphoreType.DMA,
        ],
    )
    def sc_kernel(
        cw_hbm, tf_hbm, lut_hbm, out_hbm,
        cw_v, tf_v, lut_v, key_v, out_v, sem0, sem1,
    ):
        wid = lax.axis_index("s") * _NC + lax.axis_index("c")
        base = wid * win_per_tile * 2 * _R
        pltpu.sync_copy(cw_hbm.at[pl.ds(base, win_per_tile * 2 * _R)], cw_v)
        pltpu.sync_copy(tf_hbm, tf_v)
        pltpu.sync_copy(lut_hbm, lut_v)
        sems = (sem0, sem1)
        jiota = lax.iota(jnp.int32, _R)
        woff = (jiota >> 3) * 256 + (jiota & 7) * 16

        def group_body(g, carry):
            gx = woff + g * 512
            c0 = [plsc.load_gather(cw_v, [gx + i]) for i in range(_R)]
            c1 = [plsc.load_gather(cw_v, [gx + (128 + i)]) for i in range(_R)]
            for i in range(_R):
                for j in range(_R):
                    d0 = (c0[i] - c0[j]).astype(jnp.int32) + 127
                    d1 = (c1[i] - c1[j]).astype(jnp.int32) + 127
                    a0 = plsc.load_gather(lut_v, [d0])
                    a1 = plsc.load_gather(lut_v, [d1])
                    key_v[i, j, :] = a0 + a1

            for hb in range(2):
                @pl.when(g > 0)
                def _wait_prev():
                    pltpu.make_async_copy(
                        out_v.at[hb],
                        out_hbm.at[pl.ds(hb * hh, hh), :, :, wid, :, pl.ds(0, _R)],
                        sems[hb],
                    ).wait()

                def row_body(i, rcarry):
                    for j in range(_R):
                        key = key_v[i, j, :]
                        for h in range(hh):
                            row = plsc.load_gather(
                                tf_v, [key + ((hb * hh + h) * _TABLE_ROWS)]
                            )
                            out_v[hb, h, i, j // 8, j % 8, :] = row
                    return rcarry

                lax.fori_loop(0, _R, row_body, 0)
                pltpu.async_copy(
                    out_v.at[hb],
                    out_hbm.at[
                        pl.ds(hb * hh, hh), :, :, wid, :, pl.ds(g * _R, _R)
                    ],
                    sems[hb],
                )
            return carry

        lax.fori_loop(0, n_groups, group_body, 0)
        for hb in range(2):
            pltpu.make_async_copy(
                out_v.at[hb],
                out_hbm.at[pl.ds(hb * hh, hh), :, :, wid, :, pl.ds(0, _R)],
                sems[hb],
            ).wait()

    return sc_kernel


def kernel(x, coords, relative_position_bias_table):
    B, Lc, _ = coords.shape
    H = int(np.ceil(np.sqrt(Lc)))
    H = H + ((-H) % _REGION_NUM)
    region = H // _REGION_NUM
    add_length = H * H - Lc
    if add_length > 0:
        coords = jnp.concatenate(
            [coords, jnp.zeros((B, add_length, 2), dtype=coords.dtype)], axis=1
        )
    L = H * H
    num_windows = (B * L) // region

    cw = coords.reshape(B, L // 128, 128, 2).transpose(0, 1, 3, 2).reshape(-1)
    tf = relative_position_bias_table[:_TABLE_ROWS].T.reshape(-1)
    lut = jnp.asarray(_ABS_LUT)

    out6 = _make_sc_kernel(num_windows)(cw, tf, lut)
    return out6.transpose(3, 5, 0, 1, 2, 4).reshape(
        num_windows, _NUM_HEADS, region, region
    )


# speedup vs baseline: 4.2673x; 1.1002x over previous
import functools
import math

import numpy as np
import jax
import jax.numpy as jnp
from jax import lax
from jax.experimental import pallas as pl
from jax.experimental.pallas import tpu as pltpu
from jax.experimental.pallas import tpu_sc as plsc

_NUM_HEADS = 16
_REGION_NUM = 8
_ALPHA = 1.9
_BETA = 1.9 * 4
_GAMMA = 1.9 * 6

_NC = 2
_NS = 16
_NW = _NC * _NS
_R = 16
_TABLE_ROWS = 33


def _abs_piecewise_lut() -> np.ndarray:
    lut = np.zeros(256, np.int32)
    scale = (_BETA - 2 * _ALPHA) / math.log(_GAMMA / _ALPHA)
    for a in range(0, 128):
        if a == 0:
            v = 0
        elif a * 1.0 <= _ALPHA * 2:
            v = 1
        else:
            v = int(min(round(math.log(a / _ALPHA) * scale), 16.0))
        lut[127 + a] = v
        lut[127 - a] = v
    return lut


_ABS_LUT = _abs_piecewise_lut()


def _make_sc_kernel(num_windows: int):
    win_per_tile = num_windows // _NW
    n_groups = win_per_tile // _R
    n_tw = num_windows // 128
    hh = _NUM_HEADS // 2

    mesh = plsc.VectorSubcoreMesh(core_axis_name="c", subcore_axis_name="s")

    @functools.partial(
        pl.kernel,
        out_type=jax.ShapeDtypeStruct(
            (_NUM_HEADS, _R, 2, n_tw, 8, 128), jnp.float32
        ),
        mesh=mesh,
        compiler_params=pltpu.CompilerParams(
            use_tc_tiling_on_sc=False, needs_layout_passes=False
        ),
        scratch_types=[
            pltpu.VMEM((win_per_tile * 2 * _R,), jnp.float32),
            pltpu.VMEM((_NUM_HEADS * _TABLE_ROWS,), jnp.float32),
            pltpu.VMEM((256,), jnp.int32),
            pltpu.VMEM((_R, _R, _R), jnp.int32),
            pltpu.VMEM((2, hh, _R, 2, 8, _R), jnp.float32),
            pltpu.SemaphoreType.DMA,
            pltpu.SemaphoreType.DMA,
        ],
    )
    def sc_kernel(
        cw_hbm, tf_hbm, lut_hbm, out_hbm,
        cw_v, tf_v, lut_v, key_v, out_v, sem0, sem1,
    ):
        wid = lax.axis_index("s") * _NC + lax.axis_index("c")
        base = wid * win_per_tile * 2 * _R
        pltpu.sync_copy(cw_hbm.at[pl.ds(base, win_per_tile * 2 * _R)], cw_v)
        pltpu.sync_copy(tf_hbm, tf_v)
        pltpu.sync_copy(lut_hbm, lut_v)
        sems = (sem0, sem1)
        jiota = lax.iota(jnp.int32, _R)
        woff = (jiota >> 3) * 256 + (jiota & 7) * 16

        def group_body(g, carry):
            gx = woff + g * 512
            c0 = [plsc.load_gather(cw_v, [gx + i]) for i in range(_R)]
            c1 = [plsc.load_gather(cw_v, [gx + (128 + i)]) for i in range(_R)]
            for i in range(_R):
                for j in range(_R):
                    d0 = (c0[i] - c0[j]).astype(jnp.int32) + 127
                    d1 = (c1[i] - c1[j]).astype(jnp.int32) + 127
                    a0 = plsc.load_gather(lut_v, [d0])
                    a1 = plsc.load_gather(lut_v, [d1])
                    key_v[i, j, :] = a0 + a1

            for hb in range(2):
                @pl.when(g > 0)
                def _wait_prev():
                    pltpu.make_async_copy(
                        out_v.at[hb],
                        out_hbm.at[pl.ds(hb * hh, hh), :, :, wid, :, pl.ds(0, _R)],
                        sems[hb],
                    ).wait()

                def row_body(i2, rcarry):
                    for u in range(2):
                        i = i2 * 2 + u
                        for j in range(_R):
                            key = key_v[i, j, :]
                            for h in range(hh):
                                row = plsc.load_gather(
                                    tf_v, [key + ((hb * hh + h) * _TABLE_ROWS)]
                                )
                                out_v[hb, h, i, j // 8, j % 8, :] = row
                    return rcarry

                lax.fori_loop(0, _R // 2, row_body, 0)
                pltpu.async_copy(
                    out_v.at[hb],
                    out_hbm.at[
                        pl.ds(hb * hh, hh), :, :, wid, :, pl.ds(g * _R, _R)
                    ],
                    sems[hb],
                )
            return carry

        lax.fori_loop(0, n_groups, group_body, 0)
        for hb in range(2):
            pltpu.make_async_copy(
                out_v.at[hb],
                out_hbm.at[pl.ds(hb * hh, hh), :, :, wid, :, pl.ds(0, _R)],
                sems[hb],
            ).wait()

    return sc_kernel


def kernel(x, coords, relative_position_bias_table):
    B, Lc, _ = coords.shape
    H = int(np.ceil(np.sqrt(Lc)))
    H = H + ((-H) % _REGION_NUM)
    region = H // _REGION_NUM
    add_length = H * H - Lc
    if add_length > 0:
        coords = jnp.concatenate(
            [coords, jnp.zeros((B, add_length, 2), dtype=coords.dtype)], axis=1
        )
    L = H * H
    num_windows = (B * L) // region

    cw = coords.reshape(B, L // 128, 128, 2).transpose(0, 1, 3, 2).reshape(-1)
    tf = relative_position_bias_table[:_TABLE_ROWS].T.reshape(-1)
    lut = jnp.asarray(_ABS_LUT)

    out6 = _make_sc_kernel(num_windows)(cw, tf, lut)
    return out6.transpose(3, 5, 0, 1, 2, 4).reshape(
        num_windows, _NUM_HEADS, region, region
    )
